# baseline (speedup 1.0000x reference)
import numpy as np
import jax
import jax.numpy as jnp
from jax import lax
from jax.experimental import pallas as pl
from jax.experimental.pallas import tpu as pltpu

_F, _D, _A = 50, 64, 64
_P = (_F * (_F - 1)) // 2
_PPAD = 1280

_i0, _i1 = np.triu_indices(_F, k=1)
_RS = np.zeros((_F, _PPAD), np.float32)
_RS[_i0, np.arange(_P)] = 1.0
_RS[_i1, np.arange(_P)] = 1.0

_BB = 64

_MNEG = np.full((1, _PPAD), -1e30, np.float32)
_MNEG[0, :_P] = 0.0
_MNEG_T = np.tile(_MNEG, (1, _BB))


def _afm_body(x_ref, w1e_ref, w1ehh_ref, b1_ref, w2_ref, rs_ref, rs2_ref,
              mneg_ref, o_ref):
    w1e = w1e_ref[...]
    w1ehh = w1ehh_ref[...]
    b1c = b1_ref[...]
    w2c = w2_ref[...]
    rs = rs_ref[...]
    rs2 = rs2_ref[...]
    mneg = mneg_ref[...]
    dn = (((0,), (0,)), ((), ()))
    dn_k1 = (((1,), (0,)), ((), ()))
    f32 = jnp.float32
    bf16 = jnp.bfloat16

    hs, qs = [], []
    for b in range(_BB):
        xb = x_ref[b].astype(bf16)
        s12 = lax.dot_general(xb, rs, dn, preferred_element_type=f32)
        t = s12 * s12
        t_hi = t.astype(bf16)
        t_lo = (t - t_hi.astype(f32)).astype(bf16)
        xr = xb.astype(f32)
        xsqn = xr * xr * -0.5
        xsq_hi = xsqn.astype(bf16)
        xsq_lo = (xsqn - xsq_hi.astype(f32)).astype(bf16)
        cb2 = lax.dot_general(jnp.concatenate([xsq_hi, xsq_lo], axis=0),
                              w1e, dn_k1,
                              preferred_element_type=f32)
        cb = cb2[:_F, :] + cb2[_F:, :]
        cb_hi = cb.astype(bf16)
        cb_lo = (cb - cb_hi.astype(f32)).astype(bf16)
        lhs = jnp.concatenate([w1ehh, cb_hi, cb_lo], axis=0)
        rhs = jnp.concatenate([t_hi, t_lo, rs2], axis=0)
        hq = lax.dot_general(lhs, rhs, dn, preferred_element_type=f32)
        hs.append(jnp.maximum(hq[:_A, :] + b1c, 0.0).astype(bf16))
        qs.append(hq[_A:_A + 1, :])
    h_t = jnp.concatenate(hs, axis=1)
    q = jnp.concatenate(qs, axis=1)

    logits = lax.dot_general(w2c, h_t, dn, preferred_element_type=f32)
    logits = logits + mneg

    for b in range(_BB):
        lg = logits[:, b * _PPAD:(b + 1) * _PPAD]
        qb = q[:, b * _PPAD:(b + 1) * _PPAD]
        m = jnp.max(lg, axis=1, keepdims=True)
        e = jnp.exp(lg - m)
        s = jnp.sum(e, axis=1, keepdims=True)
        num = jnp.sum(e * qb, axis=1, keepdims=True)
        o_ref[b, :, :] = num / s


@jax.jit
def _afm(inputs, W1, b1, w2, p):
    B = inputs.shape[0]
    w1e_f = jnp.concatenate([W1, p[:, None]], axis=1)
    w1e = w1e_f.astype(jnp.bfloat16)
    w1eh = (0.5 * w1e_f).astype(jnp.bfloat16)
    w1ehh = jnp.concatenate([w1eh, w1eh], axis=0)
    b1c = b1[:, None]
    w2c = w2[:, None].astype(jnp.bfloat16)
    rs = jnp.asarray(_RS).astype(jnp.bfloat16)
    rs2 = jnp.concatenate([rs, rs], axis=0)
    mneg = jnp.asarray(_MNEG_T)
    grid = (B // _BB,)
    out = pl.pallas_call(
        _afm_body,
        grid=grid,
        in_specs=[
            pl.BlockSpec((_BB, _F, _D), lambda i: (i, 0, 0)),
            pl.BlockSpec((_D, _A + 1), lambda i: (0, 0)),
            pl.BlockSpec((2 * _D, _A + 1), lambda i: (0, 0)),
            pl.BlockSpec((_A, 1), lambda i: (0, 0)),
            pl.BlockSpec((_A, 1), lambda i: (0, 0)),
            pl.BlockSpec((_F, _PPAD), lambda i: (0, 0)),
            pl.BlockSpec((2 * _F, _PPAD), lambda i: (0, 0)),
            pl.BlockSpec((1, _BB * _PPAD), lambda i: (0, 0)),
        ],
        out_specs=pl.BlockSpec((_BB, 1, 1), lambda i: (i, 0, 0)),
        out_shape=jax.ShapeDtypeStruct((B, 1, 1), jnp.float32),
        compiler_params=pltpu.CompilerParams(
            dimension_semantics=(pltpu.PARALLEL,),
            vmem_limit_bytes=56 * 1024 * 1024,
        ),
    )(inputs, w1e, w1ehh, b1c, w2c, rs, rs2, mneg)
    return out.reshape(B)


def kernel(inputs, W1, b1, w2, p):
    return _afm(inputs, W1, b1, w2, p)
